# Initial kernel scaffold; baseline (speedup 1.0000x reference)
import jax
import jax.numpy as jnp
from jax.experimental import pallas as pl


def kernel(input_ids, table):
    raise NotImplementedError("write your pallas kernel here")


# speedup vs baseline: 1.4952x; 1.4952x over previous
import functools

import jax
import jax.numpy as jnp
from jax import lax
from jax.experimental import pallas as pl
from jax.experimental.pallas import tpu as pltpu
from jax.experimental.pallas import tpu_sc as plsc

_N_WORKERS = 32
_CHUNK = 3200


@functools.lru_cache(maxsize=None)
def _make_gather(n_idx: int, dim: int):
    b_per_w = n_idx // _N_WORKERS
    n_chunks = b_per_w // _CHUNK
    mesh = plsc.VectorSubcoreMesh(core_axis_name="c", subcore_axis_name="s")

    @functools.partial(
        pl.kernel,
        mesh=mesh,
        out_type=jax.ShapeDtypeStruct((n_idx, dim), jnp.float32),
        scratch_types=[
            pltpu.VMEM((_CHUNK,), jnp.int32),
            pltpu.VMEM((_CHUNK, dim), jnp.float32),
            pltpu.SemaphoreType.DMA,
        ],
        compiler_params=pltpu.CompilerParams(use_tc_tiling_on_sc=False),
    )
    def gather_kernel(idx_hbm, table_hbm, out_hbm, idx_v, rows_v, sem):
        wid = lax.axis_index("s") * 2 + lax.axis_index("c")
        base = wid * b_per_w

        def body(g, carry):
            start = base + g * _CHUNK
            pltpu.sync_copy(idx_hbm.at[pl.ds(start, _CHUNK)], idx_v)
            pltpu.async_copy(table_hbm.at[idx_v], rows_v, sem).wait()
            pltpu.sync_copy(rows_v, out_hbm.at[pl.ds(start, _CHUNK)])
            return carry

        lax.fori_loop(0, n_chunks, body, 0)

    return gather_kernel


def kernel(input_ids, table):
    batch, hist = input_ids.shape
    dim = table.shape[1]
    ids = input_ids.reshape(-1).astype(jnp.int32)
    out = _make_gather(ids.shape[0], dim)(ids, table)
    return out.reshape(batch, hist, dim)


# speedup vs baseline: 2.0678x; 1.3829x over previous
import functools

import jax
import jax.numpy as jnp
from jax import lax
from jax.experimental import pallas as pl
from jax.experimental.pallas import tpu as pltpu
from jax.experimental.pallas import tpu_sc as plsc

_N_WORKERS = 32
_LANE = 128
_SUB = 8


@functools.lru_cache(maxsize=None)
def _make_gather(hist: int, batch: int, dim: int):
    bw = batch // _N_WORKERS
    assert bw == _LANE
    td = dim // _SUB
    n_groups = hist // 2
    pitch = _LANE + 1
    mesh = plsc.VectorSubcoreMesh(core_axis_name="c", subcore_axis_name="s")

    @functools.partial(
        pl.kernel,
        mesh=mesh,
        out_type=jax.ShapeDtypeStruct(
            (hist, td, _N_WORKERS, _SUB, _LANE), jnp.float32
        ),
        scratch_types=[
            pltpu.VMEM((hist, bw), jnp.int32),
            pltpu.VMEM((bw, dim), jnp.float32),
            pltpu.VMEM((bw, dim), jnp.float32),
            pltpu.VMEM((td, _SUB, pitch), jnp.float32),
            pltpu.VMEM((td, _SUB, pitch), jnp.float32),
        ]
        + [pltpu.SemaphoreType.DMA] * 5,
        compiler_params=pltpu.CompilerParams(
            use_tc_tiling_on_sc=False, needs_layout_passes=False
        ),
    )
    def gather_kernel(ids_hbm, table_hbm, out_hbm, idx_v, rows0, rows1,
                      tr0, tr1, sem_i, g0, g1, o0, o1):
        wid = lax.axis_index("s") * 2 + lax.axis_index("c")
        b0 = wid * bw

        lane = lax.iota(jnp.int32, 16)
        td_lo, di_lo = lane // _SUB, lane % _SUB
        td_hi = td_lo + 16 // _SUB

        def transpose(rows, tr):
            for r in range(bw):
                rv = jnp.full((16,), r, jnp.int32)
                plsc.store_scatter(tr, [td_lo, di_lo, rv], rows[r, 0:16])
                plsc.store_scatter(tr, [td_hi, di_lo, rv], rows[r, 16:32])

        def gather_copy(h, rows, sem):
            return pltpu.make_async_copy(
                table_hbm.at[idx_v.at[h]], rows, sem
            )

        def out_copy(h, tr, sem):
            return pltpu.make_async_copy(
                tr.at[:, :, 0:bw], out_hbm.at[h, :, wid, :, :], sem
            )

        pltpu.make_async_copy(
            ids_hbm.at[:, pl.ds(b0, bw)], idx_v, sem_i
        ).start()
        pltpu.make_async_copy(
            ids_hbm.at[:, pl.ds(b0, bw)], idx_v, sem_i
        ).wait()
        gather_copy(0, rows0, g0).start()

        def group(g, carry):
            h0 = 2 * g
            h1 = h0 + 1
            gather_copy(h0, rows0, g0).wait()
            gather_copy(h1, rows1, g1).start()

            @pl.when(g > 0)
            def _():
                out_copy(h0, tr0, o0).wait()

            transpose(rows0, tr0)
            out_copy(h0, tr0, o0).start()

            @pl.when(g < n_groups - 1)
            def _():
                gather_copy(h0 + 2, rows0, g0).start()

            gather_copy(h1, rows1, g1).wait()

            @pl.when(g > 0)
            def _():
                out_copy(h1, tr1, o1).wait()

            transpose(rows1, tr1)
            out_copy(h1, tr1, o1).start()
            return carry

        lax.fori_loop(0, n_groups, group, 0)
        out_copy(hist - 2, tr0, o0).wait()
        out_copy(hist - 1, tr1, o1).wait()

    return gather_kernel


def kernel(input_ids, table):
    batch, hist = input_ids.shape
    dim = table.shape[1]
    ids_t = jnp.transpose(input_ids).astype(jnp.int32)
    out_tiles = _make_gather(hist, batch, dim)(ids_t, table)
    return jnp.transpose(out_tiles, (2, 4, 0, 1, 3)).reshape(batch, hist, dim)
